# baseline (speedup 1.0000x reference)
import functools

import jax
import jax.numpy as jnp
from jax import lax
from jax.experimental import pallas as pl
from jax.experimental.pallas import tpu as pltpu
from jax.experimental.pallas import tpu_sc as plsc

NUM_POSITIONS = 8192
EMBEDDING_DIM = 2048
TOTAL = 4 * 8192
PER_CORE = TOTAL // 2
CHUNK = 128
NSEM = 8


def _scs_body(idx_hbm, table_hbm, out_hbm, idx_s, sems):
    cid = lax.axis_index("c")
    base = cid * PER_CORE

    def chunk(ci, _):
        cbase = pl.multiple_of(base + ci * CHUNK, 8)
        pltpu.sync_copy(idx_hbm.at[pl.ds(cbase, CHUNK)], idx_s)

        def body(j, _):
            row = idx_s[j]
            g = ci * CHUNK + j

            @pl.when(g >= NSEM)
            def _():
                pltpu.make_async_copy(
                    table_hbm.at[pl.ds(0, 1)], out_hbm.at[pl.ds(0, 1)],
                    sems.at[lax.rem(g, NSEM)]
                ).wait()

            pltpu.make_async_copy(
                table_hbm.at[pl.ds(row, 1)], out_hbm.at[pl.ds(cbase + j, 1)],
                sems.at[lax.rem(g, NSEM)]
            ).start()
            return 0

        lax.fori_loop(0, CHUNK, body, 0)
        return 0

    lax.fori_loop(0, PER_CORE // CHUNK, chunk, 0)

    def drain(j, _):
        pltpu.make_async_copy(
            table_hbm.at[pl.ds(0, 1)], out_hbm.at[pl.ds(0, 1)],
            sems.at[lax.rem(j, NSEM)]
        ).wait()
        return 0

    lax.fori_loop(0, NSEM, drain, 0)


@functools.partial(
    pl.kernel,
    out_type=jax.ShapeDtypeStruct((TOTAL, EMBEDDING_DIM), jnp.float32),
    mesh=plsc.ScalarSubcoreMesh(axis_name="c", num_cores=2),
    scratch_types=[
        pltpu.SMEM((CHUNK,), jnp.int32),
        pltpu.SemaphoreType.DMA((NSEM,)),
    ],
)
def _emb(idx_hbm, table_hbm, out_hbm, idx_s, sems):
    _scs_body(idx_hbm, table_hbm, out_hbm, idx_s, sems)


def kernel(positions, weight):
    flat = positions.reshape(-1)
    out = _emb(flat, weight)
    return out.reshape(positions.shape + (weight.shape[1],))


# speedup vs baseline: 39.7452x; 39.7452x over previous
import functools

import jax
import jax.numpy as jnp
from jax import lax
from jax.experimental import pallas as pl
from jax.experimental.pallas import tpu as pltpu
from jax.experimental.pallas import tpu_sc as plsc

NUM_POSITIONS = 8192
EMBEDDING_DIM = 2048
TOTAL = 4 * 8192

NUM_WORKERS = 32
B_PER_W = TOTAL // NUM_WORKERS
K = 16
NBUF = 3
G = NBUF - 1
STEPS = B_PER_W // K


def _emb_body(idx_hbm, table_hbm, out_hbm, idx_v, rows_v, gsems, osems):
    nc = plsc.get_sparse_core_info().num_cores
    wid = lax.axis_index("s") * nc + lax.axis_index("c")
    base = wid * B_PER_W

    pltpu.sync_copy(idx_hbm.at[pl.ds(base, B_PER_W)], idx_v)

    def gather(step, buf):
        off = pl.multiple_of(step * K, 8)
        return pltpu.make_async_copy(
            table_hbm.at[idx_v.at[pl.ds(off, K)]], rows_v.at[buf], gsems.at[buf]
        )

    def write(step, buf):
        off = pl.multiple_of(base + step * K, 8)
        return pltpu.make_async_copy(
            rows_v.at[buf], out_hbm.at[pl.ds(off, K)], osems.at[buf]
        )

    for b in range(G):
        gather(b, b).start()

    def body(i, _):
        buf = lax.rem(i, NBUF)

        @pl.when(i + G < STEPS)
        def _():
            nbuf = lax.rem(i + G, NBUF)

            @pl.when(i >= 1)
            def _():
                write(i - 1, nbuf).wait()

            gather(i + G, nbuf).start()

        gather(i, buf).wait()
        write(i, buf).start()
        return 0

    lax.fori_loop(0, STEPS, body, 0)

    for j in range(STEPS - G - 1, STEPS):
        write(j, j % NBUF).wait()


@functools.partial(
    pl.kernel,
    out_type=jax.ShapeDtypeStruct((TOTAL, EMBEDDING_DIM), jnp.float32),
    mesh=plsc.VectorSubcoreMesh(core_axis_name="c", subcore_axis_name="s"),
    scratch_types=[
        pltpu.VMEM((B_PER_W,), jnp.int32),
        pltpu.VMEM((NBUF, K, EMBEDDING_DIM), jnp.float32),
        pltpu.SemaphoreType.DMA((NBUF,)),
        pltpu.SemaphoreType.DMA((NBUF,)),
    ],
)
def _emb(idx_hbm, table_hbm, out_hbm, idx_v, rows_v, gsems, osems):
    _emb_body(idx_hbm, table_hbm, out_hbm, idx_v, rows_v, gsems, osems)


def kernel(positions, weight):
    flat = positions.reshape(-1)
    out = _emb(flat, weight)
    return out.reshape(positions.shape + (weight.shape[1],))
